# baseline (speedup 1.0000x reference)
import functools

import jax
import jax.numpy as jnp
import numpy as np
from jax.experimental import pallas as pl
from jax.experimental.pallas import tpu as pltpu

_NUM_TABLE = 32
_CODE_LENGTH = 16
_HIDDEN = 2048
_TOTAL = _NUM_TABLE * _CODE_LENGTH
_BM = 256


def _fused_kernel(x_ref, w_ref, b_ref, selc_ref, sele_ref, code_ref, score_ref):
    z = jnp.dot(x_ref[...], w_ref[...], preferred_element_type=jnp.float32)
    z = z + b_ref[...]
    bits = (z > 0).astype(jnp.float32)
    codef = jax.lax.dot(bits, selc_ref[...],
                        preferred_element_type=jnp.float32)
    code_ref[...] = codef.astype(jnp.int32)
    za = jnp.abs(z)
    p = za * pltpu.roll(za, _TOTAL - 1, 1)
    p = p * pltpu.roll(p, _TOTAL - 2, 1)
    p = p * pltpu.roll(p, _TOTAL - 4, 1)
    p = p * pltpu.roll(p, _TOTAL - 8, 1)
    score_ref[...] = jax.lax.dot(p, sele_ref[...],
                                 preferred_element_type=jnp.float32)


@functools.partial(jax.jit, static_argnames=("interpret",))
def kernel(x, W, b, interpret=False):
    Bsz = x.shape[0]
    d = np.arange(_TOTAL)
    sel_code = np.where((d[:, None] // _CODE_LENGTH) == np.arange(_NUM_TABLE)[None, :],
                        (2.0 ** (d % _CODE_LENGTH))[:, None], 0.0).astype(np.float32)
    sel_ext = (d[:, None] == (_CODE_LENGTH * np.arange(_NUM_TABLE))[None, :]
               ).astype(np.float32)
    grid = (Bsz // _BM,)
    code, score = pl.pallas_call(
        _fused_kernel,
        grid=grid,
        in_specs=[
            pl.BlockSpec((_BM, _HIDDEN), lambda i: (i, 0)),
            pl.BlockSpec((_HIDDEN, _TOTAL), lambda i: (0, 0)),
            pl.BlockSpec((1, _TOTAL), lambda i: (0, 0)),
            pl.BlockSpec((_TOTAL, _NUM_TABLE), lambda i: (0, 0)),
            pl.BlockSpec((_TOTAL, _NUM_TABLE), lambda i: (0, 0)),
        ],
        out_specs=[
            pl.BlockSpec((_BM, _NUM_TABLE), lambda i: (i, 0)),
            pl.BlockSpec((_BM, _NUM_TABLE), lambda i: (i, 0)),
        ],
        out_shape=[
            jax.ShapeDtypeStruct((Bsz, _NUM_TABLE), jnp.int32),
            jax.ShapeDtypeStruct((Bsz, _NUM_TABLE), jnp.float32),
        ],
        compiler_params=pltpu.CompilerParams(
            dimension_semantics=("parallel",)),
        interpret=interpret,
    )(x, W, b.reshape(1, _TOTAL), jnp.asarray(sel_code), jnp.asarray(sel_ext))
    return (code, score)


# speedup vs baseline: 1.2484x; 1.2484x over previous
import functools

import jax
import jax.numpy as jnp
import numpy as np
from jax.experimental import pallas as pl
from jax.experimental.pallas import tpu as pltpu

_NUM_TABLE = 32
_CODE_LENGTH = 16
_HIDDEN = 2048
_TOTAL = _NUM_TABLE * _CODE_LENGTH
_BM = 1024


def _fused_kernel(x_ref, w_ref, b_ref, selc_ref, sele_ref, code_ref, score_ref):
    z = jnp.dot(x_ref[...], w_ref[...], preferred_element_type=jnp.float32)
    z = z + b_ref[...]
    bits = (z > 0).astype(jnp.float32)
    codef = jax.lax.dot(bits, selc_ref[...],
                        preferred_element_type=jnp.float32)
    code_ref[...] = codef.astype(jnp.int32)
    za = jnp.abs(z)
    p = za * pltpu.roll(za, _TOTAL - 1, 1)
    p = p * pltpu.roll(p, _TOTAL - 2, 1)
    p = p * pltpu.roll(p, _TOTAL - 4, 1)
    p = p * pltpu.roll(p, _TOTAL - 8, 1)
    score_ref[...] = jax.lax.dot(p, sele_ref[...],
                                 preferred_element_type=jnp.float32)


@functools.partial(jax.jit, static_argnames=("interpret",))
def kernel(x, W, b, interpret=False):
    Bsz = x.shape[0]
    d = np.arange(_TOTAL)
    sel_code = np.where((d[:, None] // _CODE_LENGTH) == np.arange(_NUM_TABLE)[None, :],
                        (2.0 ** (d % _CODE_LENGTH))[:, None], 0.0).astype(np.float32)
    sel_ext = (d[:, None] == (_CODE_LENGTH * np.arange(_NUM_TABLE))[None, :]
               ).astype(np.float32)
    grid = (Bsz // _BM,)
    code, score = pl.pallas_call(
        _fused_kernel,
        grid=grid,
        in_specs=[
            pl.BlockSpec((_BM, _HIDDEN), lambda i: (i, 0)),
            pl.BlockSpec((_HIDDEN, _TOTAL), lambda i: (0, 0)),
            pl.BlockSpec((1, _TOTAL), lambda i: (0, 0)),
            pl.BlockSpec((_TOTAL, _NUM_TABLE), lambda i: (0, 0)),
            pl.BlockSpec((_TOTAL, _NUM_TABLE), lambda i: (0, 0)),
        ],
        out_specs=[
            pl.BlockSpec((_BM, _NUM_TABLE), lambda i: (i, 0)),
            pl.BlockSpec((_BM, _NUM_TABLE), lambda i: (i, 0)),
        ],
        out_shape=[
            jax.ShapeDtypeStruct((Bsz, _NUM_TABLE), jnp.int32),
            jax.ShapeDtypeStruct((Bsz, _NUM_TABLE), jnp.float32),
        ],
        compiler_params=pltpu.CompilerParams(
            dimension_semantics=("parallel",)),
        interpret=interpret,
    )(x, W, b.reshape(1, _TOTAL), jnp.asarray(sel_code), jnp.asarray(sel_ext))
    return (code, score)
